# baseline (speedup 1.0000x reference)
import math

import jax
import jax.numpy as jnp
from jax.experimental import pallas as pl
from jax.experimental.pallas import tpu as pltpu

B = 2048
D = 192
DO = 96
DM = 96
H = 64
C = 4
M = 8
L = 256 * 64

BLK = 64
NBLK = B // BLK
LOG2PI = math.log(2.0 * math.pi)


def _te_block(inv_ref, oh_ref, xo_ref, xm_ref, nu_ref, tnu_ref, cinu_ref,
              ciwt_ref, wo_ref, ww_ref, tf_ref, aux_ref, ldets_ref, lp_ref,
              ci_ref, cicm_ref, out_ref, y1_ref, y2_ref, t_ref):
    inv_cap = inv_ref[0]
    xo = xo_ref[:]
    xm = xm_ref[:]
    nu_flat = nu_ref[:].reshape(BLK * C, DO)

    y1_ref[:] = jnp.zeros((BLK, DO), jnp.float32)
    y2_ref[:] = jnp.zeros((BLK, DM), jnp.float32)
    t_ref[:] = jnp.zeros((BLK * C, DM), jnp.float32)

    nbf = oh_ref[:]

    def h_body(h, carry):
        colmask = (nbf == h.astype(jnp.float32)).astype(jnp.float32)
        present = jnp.sum(colmask) > 0.5

        @pl.when(present)
        def _():
            ci_h = ci_ref[h]
            cicm_h = cicm_ref[h]
            y1_ref[:] += colmask * jnp.dot(
                xo, ci_h, preferred_element_type=jnp.float32)
            y2_ref[:] += colmask * jnp.dot(
                xo, cicm_h, preferred_element_type=jnp.float32)
            mask4 = jnp.broadcast_to(
                colmask[:, None, :], (BLK, C, 1)).reshape(BLK * C, 1)
            t_ref[:] += mask4 * jnp.dot(
                nu_flat, cicm_h, preferred_element_type=jnp.float32)

        return carry

    jax.lax.fori_loop(0, H, h_body, 0)

    y1 = y1_ref[:]
    y2 = y2_ref[:]
    t4 = t_ref[:].reshape(BLK, C, DM)

    ld = aux_ref[:, 0]
    nob = aux_ref[:, 1]
    nll = aux_ref[:, 2]
    xCx = jnp.sum(y1 * xo, axis=1)
    base = -0.5 * (ld + nob * LOG2PI) - nll

    nu = nu_ref[:]
    dx = xo[:, None, :] - nu
    mahal = xCx[:, None] - 2.0 * jnp.sum(xo[:, None, :] * cinu_ref[:], axis=2) \
        + jnp.sum(nu * cinu_ref[:], axis=2)

    p = jnp.sum(ciwt_ref[:] * dx[:, :, None, :], axis=3)

    jf = jax.lax.broadcasted_iota(jnp.int32, (M, M * M), 1)
    mrow = jax.lax.broadcasted_iota(jnp.int32, (M, M * M), 0)
    ra = (jf // M == mrow).astype(jnp.float32)
    rb = (jf % M == mrow).astype(jnp.float32)
    p2 = p.reshape(BLK * C, M)
    pa = jnp.dot(p2, ra, preferred_element_type=jnp.float32)
    pb = jnp.dot(p2, rb, preferred_element_type=jnp.float32)
    corr = jnp.sum(tf_ref[:].reshape(BLK * C, M * M) * pa * pb,
                   axis=1).reshape(BLK, C)

    r = xm[:, None, :] - tnu_ref[:] - y2[:, None, :] + t4
    rsq = jnp.sum(r * r, axis=2)

    jg = jax.lax.broadcasted_iota(jnp.int32, (DO, DO * M), 1)
    drow = jax.lax.broadcasted_iota(jnp.int32, (DO, DO * M), 0)
    rep_d = (jg // M == drow).astype(jnp.float32)
    jm = jax.lax.broadcasted_iota(jnp.int32, (M, DO * M), 1)
    mrow2 = jax.lax.broadcasted_iota(jnp.int32, (M, DO * M), 0)
    rep_m = (jm % M == mrow2).astype(jnp.float32)
    dxrep = jnp.dot(dx.reshape(BLK * C, DO), rep_d,
                    preferred_element_type=jnp.float32)
    prep = jnp.dot(p2, rep_m, preferred_element_type=jnp.float32)
    dp = dxrep * prep
    wq_p = jnp.sum(ww_ref[:].reshape(BLK * C, DO * M) * dp,
                   axis=1).reshape(BLK, C)
    wo_p = jnp.sum(wo_ref[:].reshape(BLK * C, DO * M) * dp,
                   axis=1).reshape(BLK, C)

    lls = base[:, None] - 0.5 * (ldets_ref[:] + mahal - corr) + lp_ref[:]
    lls = lls - 0.5 * inv_cap * rsq
    lls = lls + wq_p + 0.01 * wo_p
    out_ref[:] = lls


def kernel(batch_indices, features, neighborhood_ids, candidates,
           unit_neighb_lut, Coo_logdet, Coo_inv, Coinv_Com, obs_ix, miss_ix,
           nobs, log_proportions, nu, tnu, Wobs, Cooinv_nu, obs_logdets,
           Cobsinv_WobsT, T, W_WCC, inv_cap, noise_logliks):
    nb0 = neighborhood_ids[batch_indices]
    oh0 = jax.nn.one_hot(nb0, H, dtype=jnp.float32)

    iota_b = jnp.arange(B, dtype=jnp.int32)
    tril = (iota_b[:, None] >= iota_b[None, :]).astype(jnp.float32)
    cum = jnp.dot(tril, oh0, preferred_element_type=jnp.float32)
    rank = jnp.sum(cum * oh0, axis=1).astype(jnp.int32) - 1
    cnt = cum[-1].astype(jnp.int32)
    off = jnp.concatenate([jnp.zeros((1,), jnp.int32),
                           jnp.cumsum(cnt)[:-1]])
    pos = off[nb0] + rank
    inv = jnp.zeros((B,), jnp.int32).at[pos].set(iota_b)

    bi = batch_indices[inv]
    nb = nb0[inv]
    nbf_col = nb.astype(jnp.float32)[:, None]
    cand = candidates[bi]
    lut = unit_neighb_lut[cand, nb[:, None]]

    x = features[bi]
    xo = jnp.take_along_axis(x, obs_ix[nb], axis=1)
    xm = jnp.take_along_axis(x, miss_ix[nb], axis=1)

    nu_b = nu[lut]
    tnu_b = tnu[lut]
    cinu_b = Cooinv_nu[lut]
    ciwt_b = Cobsinv_WobsT[lut]
    wo_b = Wobs.reshape(L, DO * M)[lut]
    ww_b = W_WCC.reshape(L, DO * M)[lut]
    tf_b = T.reshape(L, M * M)[lut]

    ld = Coo_logdet[nb]
    ldets = obs_logdets[lut]
    lp = log_proportions[cand]
    nob = nobs[nb].astype(jnp.float32)
    nll = noise_logliks[bi]
    aux = jnp.stack([ld, nob, nll, jnp.zeros_like(ld)], axis=1)
    inv_arr = jnp.reshape(inv_cap, (1,)).astype(jnp.float32)

    spec = lambda bs, im: pl.BlockSpec(bs, im)
    in_specs = [
            pl.BlockSpec(memory_space=pltpu.SMEM),
            spec((BLK, 1), lambda i: (i, 0)),
            spec((BLK, DO), lambda i: (i, 0)),
            spec((BLK, DM), lambda i: (i, 0)),
            spec((BLK, C, DO), lambda i: (i, 0, 0)),
            spec((BLK, C, DM), lambda i: (i, 0, 0)),
            spec((BLK, C, DO), lambda i: (i, 0, 0)),
            spec((BLK, C, M, DO), lambda i: (i, 0, 0, 0)),
            spec((BLK, C, DO * M), lambda i: (i, 0, 0)),
            spec((BLK, C, DO * M), lambda i: (i, 0, 0)),
            spec((BLK, C, M * M), lambda i: (i, 0, 0)),
            spec((BLK, 4), lambda i: (i, 0)),
            spec((BLK, C), lambda i: (i, 0)),
            spec((BLK, C), lambda i: (i, 0)),
            spec((H, DO, DO), lambda i: (0, 0, 0)),
            spec((H, DO, DM), lambda i: (0, 0, 0)),
    ]
    lls_sorted = pl.pallas_call(
        _te_block,
        grid=(NBLK,),
        in_specs=in_specs,
        out_specs=spec((BLK, C), lambda i: (i, 0)),
        out_shape=jax.ShapeDtypeStruct((B, C), jnp.float32),
        scratch_shapes=[
            pltpu.VMEM((BLK, DO), jnp.float32),
            pltpu.VMEM((BLK, DM), jnp.float32),
            pltpu.VMEM((BLK * C, DM), jnp.float32),
        ],
    )(inv_arr, nbf_col, xo, xm, nu_b, tnu_b, cinu_b, ciwt_b, wo_b, ww_b, tf_b,
      aux, ldets, lp, Coo_inv, Coinv_Com)
    return lls_sorted[pos]


# speedup vs baseline: 1.5350x; 1.5350x over previous
import math

import jax
import jax.numpy as jnp
from jax.experimental import pallas as pl
from jax.experimental.pallas import tpu as pltpu

B = 2048
D = 192
DO = 96
DM = 96
H = 64
C = 4
M = 8
L = 256 * 64

BLK = 64
NBLK = B // BLK
LOG2PI = math.log(2.0 * math.pi)


def _te_block(inv_ref, oh_ref, xo_ref, xm_ref, nu_ref, tnu_ref, cinu_ref,
              ciwt_ref, ww_ref, tf_ref, aux_ref, ldets_ref, lp_ref,
              ci_ref, cicm_ref, out_ref, y1_ref, y2_ref, t_ref):
    inv_cap = inv_ref[0]
    xo = xo_ref[:]
    xm = xm_ref[:]
    nu_flat = nu_ref[:].reshape(BLK * C, DO)

    y1_ref[:] = jnp.zeros((BLK, DO), jnp.float32)
    y2_ref[:] = jnp.zeros((BLK, DM), jnp.float32)
    t_ref[:] = jnp.zeros((BLK * C, DM), jnp.float32)

    nbf = oh_ref[:]

    def h_body(h, carry):
        colmask = (nbf == h.astype(jnp.float32)).astype(jnp.float32)
        present = jnp.sum(colmask) > 0.5

        @pl.when(present)
        def _():
            ci_h = ci_ref[h]
            cicm_h = cicm_ref[h]
            y1_ref[:] += colmask * jnp.dot(
                xo, ci_h, preferred_element_type=jnp.float32)
            y2_ref[:] += colmask * jnp.dot(
                xo, cicm_h, preferred_element_type=jnp.float32)
            mask4 = jnp.broadcast_to(
                colmask[:, None, :], (BLK, C, 1)).reshape(BLK * C, 1)
            t_ref[:] += mask4 * jnp.dot(
                nu_flat, cicm_h, preferred_element_type=jnp.float32)

        return carry

    jax.lax.fori_loop(0, H, h_body, 0)

    y1 = y1_ref[:]
    y2 = y2_ref[:]
    t4 = t_ref[:].reshape(BLK, C, DM)

    ld = aux_ref[:, 0]
    nob = aux_ref[:, 1]
    nll = aux_ref[:, 2]
    xCx = jnp.sum(y1 * xo, axis=1)
    base = -0.5 * (ld + nob * LOG2PI) - nll

    nu = nu_ref[:]
    dx = xo[:, None, :] - nu
    mahal = xCx[:, None] - 2.0 * jnp.sum(xo[:, None, :] * cinu_ref[:], axis=2) \
        + jnp.sum(nu * cinu_ref[:], axis=2)

    p = jnp.sum(ciwt_ref[:] * dx[:, :, None, :], axis=3)

    jf = jax.lax.broadcasted_iota(jnp.int32, (M, M * M), 1)
    mrow = jax.lax.broadcasted_iota(jnp.int32, (M, M * M), 0)
    ra = (jf // M == mrow).astype(jnp.float32)
    rb = (jf % M == mrow).astype(jnp.float32)
    p2 = p.reshape(BLK * C, M)
    pa = jnp.dot(p2, ra, preferred_element_type=jnp.float32)
    pb = jnp.dot(p2, rb, preferred_element_type=jnp.float32)
    corr = jnp.sum(tf_ref[:].reshape(BLK * C, M * M) * pa * pb,
                   axis=1).reshape(BLK, C)

    r = xm[:, None, :] - tnu_ref[:] - y2[:, None, :] + t4
    rsq = jnp.sum(r * r, axis=2)

    jg = jax.lax.broadcasted_iota(jnp.int32, (DO, DO * M), 1)
    drow = jax.lax.broadcasted_iota(jnp.int32, (DO, DO * M), 0)
    rep_d = (jg // M == drow).astype(jnp.float32)
    jm = jax.lax.broadcasted_iota(jnp.int32, (M, DO * M), 1)
    mrow2 = jax.lax.broadcasted_iota(jnp.int32, (M, DO * M), 0)
    rep_m = (jm % M == mrow2).astype(jnp.float32)
    dxrep = jnp.dot(dx.reshape(BLK * C, DO), rep_d,
                    preferred_element_type=jnp.float32)
    prep = jnp.dot(p2, rep_m, preferred_element_type=jnp.float32)
    dp = dxrep * prep
    wq_p = jnp.sum(ww_ref[:].reshape(BLK * C, DO * M) * dp,
                   axis=1).reshape(BLK, C)

    lls = base[:, None] - 0.5 * (ldets_ref[:] + mahal - corr) + lp_ref[:]
    lls = lls - 0.5 * inv_cap * rsq
    lls = lls + wq_p
    out_ref[:] = lls


def kernel(batch_indices, features, neighborhood_ids, candidates,
           unit_neighb_lut, Coo_logdet, Coo_inv, Coinv_Com, obs_ix, miss_ix,
           nobs, log_proportions, nu, tnu, Wobs, Cooinv_nu, obs_logdets,
           Cobsinv_WobsT, T, W_WCC, inv_cap, noise_logliks):
    nb0 = neighborhood_ids[batch_indices]
    oh0 = jax.nn.one_hot(nb0, H, dtype=jnp.float32)

    iota_b = jnp.arange(B, dtype=jnp.int32)
    tril = (iota_b[:, None] >= iota_b[None, :]).astype(jnp.float32)
    cum = jnp.dot(tril, oh0, preferred_element_type=jnp.float32)
    rank = jnp.sum(cum * oh0, axis=1).astype(jnp.int32) - 1
    cnt = cum[-1].astype(jnp.int32)
    off = jnp.concatenate([jnp.zeros((1,), jnp.int32),
                           jnp.cumsum(cnt)[:-1]])
    pos = off[nb0] + rank
    inv = jnp.zeros((B,), jnp.int32).at[pos].set(iota_b)

    bi = batch_indices[inv]
    nb = nb0[inv]
    nbf_col = nb.astype(jnp.float32)[:, None]
    cand = candidates[bi]
    lut = unit_neighb_lut[cand, nb[:, None]]

    ft = jnp.swapaxes(features, 0, 1)
    xo = ft[obs_ix[nb], bi[:, None]]
    xm = ft[miss_ix[nb], bi[:, None]]

    nu_b = nu[lut]
    tnu_b = tnu[lut]
    cinu_b = Cooinv_nu[lut]
    ciwt_b = Cobsinv_WobsT[lut]
    comb = (W_WCC + 0.01 * Wobs).reshape(L, DO * M)
    ww_b = comb[lut]
    tf_b = T.reshape(L, M * M)[lut]

    ld = Coo_logdet[nb]
    ldets = obs_logdets[lut]
    lp = log_proportions[cand]
    nob = nobs[nb].astype(jnp.float32)
    nll = noise_logliks[bi]
    aux = jnp.stack([ld, nob, nll, jnp.zeros_like(ld)], axis=1)
    inv_arr = jnp.reshape(inv_cap, (1,)).astype(jnp.float32)

    spec = lambda bs, im: pl.BlockSpec(bs, im)
    in_specs = [
            pl.BlockSpec(memory_space=pltpu.SMEM),
            spec((BLK, 1), lambda i: (i, 0)),
            spec((BLK, DO), lambda i: (i, 0)),
            spec((BLK, DM), lambda i: (i, 0)),
            spec((BLK, C, DO), lambda i: (i, 0, 0)),
            spec((BLK, C, DM), lambda i: (i, 0, 0)),
            spec((BLK, C, DO), lambda i: (i, 0, 0)),
            spec((BLK, C, M, DO), lambda i: (i, 0, 0, 0)),
            spec((BLK, C, DO * M), lambda i: (i, 0, 0)),
            spec((BLK, C, M * M), lambda i: (i, 0, 0)),
            spec((BLK, 4), lambda i: (i, 0)),
            spec((BLK, C), lambda i: (i, 0)),
            spec((BLK, C), lambda i: (i, 0)),
            spec((H, DO, DO), lambda i: (0, 0, 0)),
            spec((H, DO, DM), lambda i: (0, 0, 0)),
    ]
    lls_sorted = pl.pallas_call(
        _te_block,
        grid=(NBLK,),
        in_specs=in_specs,
        out_specs=spec((BLK, C), lambda i: (i, 0)),
        out_shape=jax.ShapeDtypeStruct((B, C), jnp.float32),
        scratch_shapes=[
            pltpu.VMEM((BLK, DO), jnp.float32),
            pltpu.VMEM((BLK, DM), jnp.float32),
            pltpu.VMEM((BLK * C, DM), jnp.float32),
        ],
    )(inv_arr, nbf_col, xo, xm, nu_b, tnu_b, cinu_b, ciwt_b, ww_b, tf_b,
      aux, ldets, lp, Coo_inv, Coinv_Com)
    return lls_sorted[pos]


# speedup vs baseline: 1.8255x; 1.1892x over previous
import math

import jax
import jax.numpy as jnp
from jax.experimental import pallas as pl
from jax.experimental.pallas import tpu as pltpu

B = 2048
D = 192
DO = 96
DM = 96
H = 64
C = 4
M = 8
L = 256 * 64

BLK = 128
NBLK = B // BLK
LOG2PI = math.log(2.0 * math.pi)


def _te_block(inv_ref, oh_ref, xo_ref, xm_ref, nu_ref, tnu_ref, cinu_ref,
              ciwt_ref, ww_ref, tf_ref, aux_ref, ldets_ref, lp_ref,
              ci_ref, cicm_ref, out_ref, y1_ref, y2_ref, t_ref):
    inv_cap = inv_ref[0]
    xo = xo_ref[:]
    xm = xm_ref[:]
    nu_flat = nu_ref[:].reshape(BLK * C, DO)

    y1_ref[:] = jnp.zeros((BLK, DO), jnp.float32)
    y2_ref[:] = jnp.zeros((BLK, DM), jnp.float32)
    t_ref[:] = jnp.zeros((BLK * C, DM), jnp.float32)

    nbf = oh_ref[:]

    def h_body(h, carry):
        colmask = (nbf == h.astype(jnp.float32)).astype(jnp.float32)
        present = jnp.sum(colmask) > 0.5

        @pl.when(present)
        def _():
            ci_h = ci_ref[h]
            cicm_h = cicm_ref[h]
            y1_ref[:] += colmask * jnp.dot(
                xo, ci_h, preferred_element_type=jnp.float32)
            y2_ref[:] += colmask * jnp.dot(
                xo, cicm_h, preferred_element_type=jnp.float32)
            mask4 = jnp.broadcast_to(
                colmask[:, None, :], (BLK, C, 1)).reshape(BLK * C, 1)
            t_ref[:] += mask4 * jnp.dot(
                nu_flat, cicm_h, preferred_element_type=jnp.float32)

        return carry

    jax.lax.fori_loop(0, H, h_body, 0)

    y1 = y1_ref[:]
    y2 = y2_ref[:]
    t4 = t_ref[:].reshape(BLK, C, DM)

    ld = aux_ref[:, 0]
    nob = aux_ref[:, 1]
    nll = aux_ref[:, 2]
    xCx = jnp.sum(y1 * xo, axis=1)
    base = -0.5 * (ld + nob * LOG2PI) - nll

    nu = nu_ref[:]
    dx = xo[:, None, :] - nu
    mahal = xCx[:, None] - 2.0 * jnp.sum(xo[:, None, :] * cinu_ref[:], axis=2) \
        + jnp.sum(nu * cinu_ref[:], axis=2)

    p = jnp.sum(ciwt_ref[:] * dx[:, :, None, :], axis=3)

    jf = jax.lax.broadcasted_iota(jnp.int32, (M, M * M), 1)
    mrow = jax.lax.broadcasted_iota(jnp.int32, (M, M * M), 0)
    ra = (jf // M == mrow).astype(jnp.float32)
    rb = (jf % M == mrow).astype(jnp.float32)
    p2 = p.reshape(BLK * C, M)
    pa = jnp.dot(p2, ra, preferred_element_type=jnp.float32)
    pb = jnp.dot(p2, rb, preferred_element_type=jnp.float32)
    corr = jnp.sum(tf_ref[:].reshape(BLK * C, M * M) * pa * pb,
                   axis=1).reshape(BLK, C)

    r = xm[:, None, :] - tnu_ref[:] - y2[:, None, :] + t4
    rsq = jnp.sum(r * r, axis=2)

    jg = jax.lax.broadcasted_iota(jnp.int32, (DO, DO * M), 1)
    drow = jax.lax.broadcasted_iota(jnp.int32, (DO, DO * M), 0)
    rep_d = (jg // M == drow).astype(jnp.float32)
    jm = jax.lax.broadcasted_iota(jnp.int32, (M, DO * M), 1)
    mrow2 = jax.lax.broadcasted_iota(jnp.int32, (M, DO * M), 0)
    rep_m = (jm % M == mrow2).astype(jnp.float32)
    dxrep = jnp.dot(dx.reshape(BLK * C, DO), rep_d,
                    preferred_element_type=jnp.float32)
    prep = jnp.dot(p2, rep_m, preferred_element_type=jnp.float32)
    dp = dxrep * prep
    wq_p = jnp.sum(ww_ref[:].reshape(BLK * C, DO * M) * dp,
                   axis=1).reshape(BLK, C)

    lls = base[:, None] - 0.5 * (ldets_ref[:] + mahal - corr) + lp_ref[:]
    lls = lls - 0.5 * inv_cap * rsq
    lls = lls + wq_p
    out_ref[:] = lls


def kernel(batch_indices, features, neighborhood_ids, candidates,
           unit_neighb_lut, Coo_logdet, Coo_inv, Coinv_Com, obs_ix, miss_ix,
           nobs, log_proportions, nu, tnu, Wobs, Cooinv_nu, obs_logdets,
           Cobsinv_WobsT, T, W_WCC, inv_cap, noise_logliks):
    nb0 = neighborhood_ids[batch_indices]
    oh0 = jax.nn.one_hot(nb0, H, dtype=jnp.float32)

    iota_b = jnp.arange(B, dtype=jnp.int32)
    tril = (iota_b[:, None] >= iota_b[None, :]).astype(jnp.float32)
    cum = jnp.dot(tril, oh0, preferred_element_type=jnp.float32)
    rank = jnp.sum(cum * oh0, axis=1).astype(jnp.int32) - 1
    cnt = cum[-1].astype(jnp.int32)
    off = jnp.concatenate([jnp.zeros((1,), jnp.int32),
                           jnp.cumsum(cnt)[:-1]])
    pos = off[nb0] + rank
    inv = jnp.zeros((B,), jnp.int32).at[pos].set(iota_b)

    bi = batch_indices[inv]
    nb = nb0[inv]
    nbf_col = nb.astype(jnp.float32)[:, None]
    cand = candidates[bi]
    lut = unit_neighb_lut[cand, nb[:, None]]

    ft = jnp.swapaxes(features, 0, 1)
    xo = ft[obs_ix[nb], bi[:, None]]
    xm = ft[miss_ix[nb], bi[:, None]]

    nu_b = nu[lut]
    tnu_b = tnu[lut]
    cinu_b = Cooinv_nu[lut]
    ciwt_b = Cobsinv_WobsT[lut]
    comb = (W_WCC + 0.01 * Wobs).reshape(L, DO * M)
    ww_b = comb[lut]
    tf_b = T.reshape(L, M * M)[lut]

    ld = Coo_logdet[nb]
    ldets = obs_logdets[lut]
    lp = log_proportions[cand]
    nob = nobs[nb].astype(jnp.float32)
    nll = noise_logliks[bi]
    aux = jnp.stack([ld, nob, nll, jnp.zeros_like(ld)], axis=1)
    inv_arr = jnp.reshape(inv_cap, (1,)).astype(jnp.float32)

    spec = lambda bs, im: pl.BlockSpec(bs, im)
    in_specs = [
            pl.BlockSpec(memory_space=pltpu.SMEM),
            spec((BLK, 1), lambda i: (i, 0)),
            spec((BLK, DO), lambda i: (i, 0)),
            spec((BLK, DM), lambda i: (i, 0)),
            spec((BLK, C, DO), lambda i: (i, 0, 0)),
            spec((BLK, C, DM), lambda i: (i, 0, 0)),
            spec((BLK, C, DO), lambda i: (i, 0, 0)),
            spec((BLK, C, M, DO), lambda i: (i, 0, 0, 0)),
            spec((BLK, C, DO * M), lambda i: (i, 0, 0)),
            spec((BLK, C, M * M), lambda i: (i, 0, 0)),
            spec((BLK, 4), lambda i: (i, 0)),
            spec((BLK, C), lambda i: (i, 0)),
            spec((BLK, C), lambda i: (i, 0)),
            spec((H, DO, DO), lambda i: (0, 0, 0)),
            spec((H, DO, DM), lambda i: (0, 0, 0)),
    ]
    lls_sorted = pl.pallas_call(
        _te_block,
        grid=(NBLK,),
        in_specs=in_specs,
        out_specs=spec((BLK, C), lambda i: (i, 0)),
        out_shape=jax.ShapeDtypeStruct((B, C), jnp.float32),
        scratch_shapes=[
            pltpu.VMEM((BLK, DO), jnp.float32),
            pltpu.VMEM((BLK, DM), jnp.float32),
            pltpu.VMEM((BLK * C, DM), jnp.float32),
        ],
    )(inv_arr, nbf_col, xo, xm, nu_b, tnu_b, cinu_b, ciwt_b, ww_b, tf_b,
      aux, ldets, lp, Coo_inv, Coinv_Com)
    return lls_sorted[pos]


# speedup vs baseline: 1.9890x; 1.0896x over previous
import math

import jax
import jax.numpy as jnp
from jax.experimental import pallas as pl
from jax.experimental.pallas import tpu as pltpu

B = 2048
D = 192
DO = 96
DM = 96
H = 64
C = 4
M = 8
L = 256 * 64

BLK = 256
NBLK = B // BLK
LOG2PI = math.log(2.0 * math.pi)


def _te_block(inv_ref, oh_ref, xo_ref, xm_ref, nu_ref, tnu_ref, cinu_ref,
              ciwt_ref, ww_ref, tf_ref, aux_ref, ldets_ref, lp_ref,
              ci_ref, cicm_ref, out_ref, y1_ref, y2_ref, t_ref):
    inv_cap = inv_ref[0]
    xo = xo_ref[:]
    xm = xm_ref[:]
    nu_flat = nu_ref[:].reshape(BLK * C, DO)

    y1_ref[:] = jnp.zeros((BLK, DO), jnp.float32)
    y2_ref[:] = jnp.zeros((BLK, DM), jnp.float32)
    t_ref[:] = jnp.zeros((BLK * C, DM), jnp.float32)

    nbf = oh_ref[:]

    def h_body(h, carry):
        colmask = (nbf == h.astype(jnp.float32)).astype(jnp.float32)
        present = jnp.sum(colmask) > 0.5

        @pl.when(present)
        def _():
            ci_h = ci_ref[h]
            cicm_h = cicm_ref[h]
            y1_ref[:] += colmask * jnp.dot(
                xo, ci_h, preferred_element_type=jnp.float32)
            y2_ref[:] += colmask * jnp.dot(
                xo, cicm_h, preferred_element_type=jnp.float32)
            mask4 = jnp.broadcast_to(
                colmask[:, None, :], (BLK, C, 1)).reshape(BLK * C, 1)
            t_ref[:] += mask4 * jnp.dot(
                nu_flat, cicm_h, preferred_element_type=jnp.float32)

        return carry

    jax.lax.fori_loop(0, H, h_body, 0)

    y1 = y1_ref[:]
    y2 = y2_ref[:]
    t4 = t_ref[:].reshape(BLK, C, DM)

    ld = aux_ref[:, 0]
    nob = aux_ref[:, 1]
    nll = aux_ref[:, 2]
    xCx = jnp.sum(y1 * xo, axis=1)
    base = -0.5 * (ld + nob * LOG2PI) - nll

    nu = nu_ref[:]
    dx = xo[:, None, :] - nu
    mahal = xCx[:, None] - 2.0 * jnp.sum(xo[:, None, :] * cinu_ref[:], axis=2) \
        + jnp.sum(nu * cinu_ref[:], axis=2)

    p = jnp.sum(ciwt_ref[:] * dx[:, :, None, :], axis=3)

    jf = jax.lax.broadcasted_iota(jnp.int32, (M, M * M), 1)
    mrow = jax.lax.broadcasted_iota(jnp.int32, (M, M * M), 0)
    ra = (jf // M == mrow).astype(jnp.float32)
    rb = (jf % M == mrow).astype(jnp.float32)
    p2 = p.reshape(BLK * C, M)
    pa = jnp.dot(p2, ra, preferred_element_type=jnp.float32)
    pb = jnp.dot(p2, rb, preferred_element_type=jnp.float32)
    corr = jnp.sum(tf_ref[:].reshape(BLK * C, M * M) * pa * pb,
                   axis=1).reshape(BLK, C)

    r = xm[:, None, :] - tnu_ref[:] - y2[:, None, :] + t4
    rsq = jnp.sum(r * r, axis=2)

    jg = jax.lax.broadcasted_iota(jnp.int32, (DO, DO * M), 1)
    drow = jax.lax.broadcasted_iota(jnp.int32, (DO, DO * M), 0)
    rep_d = (jg // M == drow).astype(jnp.float32)
    jm = jax.lax.broadcasted_iota(jnp.int32, (M, DO * M), 1)
    mrow2 = jax.lax.broadcasted_iota(jnp.int32, (M, DO * M), 0)
    rep_m = (jm % M == mrow2).astype(jnp.float32)
    dxrep = jnp.dot(dx.reshape(BLK * C, DO), rep_d,
                    preferred_element_type=jnp.float32)
    prep = jnp.dot(p2, rep_m, preferred_element_type=jnp.float32)
    dp = dxrep * prep
    wq_p = jnp.sum(ww_ref[:].reshape(BLK * C, DO * M) * dp,
                   axis=1).reshape(BLK, C)

    lls = base[:, None] - 0.5 * (ldets_ref[:] + mahal - corr) + lp_ref[:]
    lls = lls - 0.5 * inv_cap * rsq
    lls = lls + wq_p
    out_ref[:] = lls


def kernel(batch_indices, features, neighborhood_ids, candidates,
           unit_neighb_lut, Coo_logdet, Coo_inv, Coinv_Com, obs_ix, miss_ix,
           nobs, log_proportions, nu, tnu, Wobs, Cooinv_nu, obs_logdets,
           Cobsinv_WobsT, T, W_WCC, inv_cap, noise_logliks):
    nb0 = neighborhood_ids[batch_indices]
    oh0 = jax.nn.one_hot(nb0, H, dtype=jnp.float32)

    iota_b = jnp.arange(B, dtype=jnp.int32)
    tril = (iota_b[:, None] >= iota_b[None, :]).astype(jnp.float32)
    cum = jnp.dot(tril, oh0, preferred_element_type=jnp.float32)
    rank = jnp.sum(cum * oh0, axis=1).astype(jnp.int32) - 1
    cnt = cum[-1].astype(jnp.int32)
    off = jnp.concatenate([jnp.zeros((1,), jnp.int32),
                           jnp.cumsum(cnt)[:-1]])
    pos = off[nb0] + rank
    inv = jnp.zeros((B,), jnp.int32).at[pos].set(iota_b)

    bi = batch_indices[inv]
    nb = nb0[inv]
    nbf_col = nb.astype(jnp.float32)[:, None]
    cand = candidates[bi]
    lut = unit_neighb_lut[cand, nb[:, None]]

    ft = jnp.swapaxes(features, 0, 1)
    xo = ft[obs_ix[nb], bi[:, None]]
    xm = ft[miss_ix[nb], bi[:, None]]

    nu_b = nu[lut]
    tnu_b = tnu[lut]
    cinu_b = Cooinv_nu[lut]
    ciwt_b = Cobsinv_WobsT[lut]
    comb = (W_WCC + 0.01 * Wobs).reshape(L, DO * M)
    ww_b = comb[lut]
    tf_b = T.reshape(L, M * M)[lut]

    ld = Coo_logdet[nb]
    ldets = obs_logdets[lut]
    lp = log_proportions[cand]
    nob = nobs[nb].astype(jnp.float32)
    nll = noise_logliks[bi]
    aux = jnp.stack([ld, nob, nll, jnp.zeros_like(ld)], axis=1)
    inv_arr = jnp.reshape(inv_cap, (1,)).astype(jnp.float32)

    spec = lambda bs, im: pl.BlockSpec(bs, im)
    in_specs = [
            pl.BlockSpec(memory_space=pltpu.SMEM),
            spec((BLK, 1), lambda i: (i, 0)),
            spec((BLK, DO), lambda i: (i, 0)),
            spec((BLK, DM), lambda i: (i, 0)),
            spec((BLK, C, DO), lambda i: (i, 0, 0)),
            spec((BLK, C, DM), lambda i: (i, 0, 0)),
            spec((BLK, C, DO), lambda i: (i, 0, 0)),
            spec((BLK, C, M, DO), lambda i: (i, 0, 0, 0)),
            spec((BLK, C, DO * M), lambda i: (i, 0, 0)),
            spec((BLK, C, M * M), lambda i: (i, 0, 0)),
            spec((BLK, 4), lambda i: (i, 0)),
            spec((BLK, C), lambda i: (i, 0)),
            spec((BLK, C), lambda i: (i, 0)),
            spec((H, DO, DO), lambda i: (0, 0, 0)),
            spec((H, DO, DM), lambda i: (0, 0, 0)),
    ]
    lls_sorted = pl.pallas_call(
        _te_block,
        grid=(NBLK,),
        in_specs=in_specs,
        out_specs=spec((BLK, C), lambda i: (i, 0)),
        out_shape=jax.ShapeDtypeStruct((B, C), jnp.float32),
        scratch_shapes=[
            pltpu.VMEM((BLK, DO), jnp.float32),
            pltpu.VMEM((BLK, DM), jnp.float32),
            pltpu.VMEM((BLK * C, DM), jnp.float32),
        ],
    )(inv_arr, nbf_col, xo, xm, nu_b, tnu_b, cinu_b, ciwt_b, ww_b, tf_b,
      aux, ldets, lp, Coo_inv, Coinv_Com)
    return lls_sorted[pos]


# speedup vs baseline: 2.0145x; 1.0128x over previous
import math

import jax
import jax.numpy as jnp
from jax.experimental import pallas as pl
from jax.experimental.pallas import tpu as pltpu

B = 2048
D = 192
DO = 96
DM = 96
H = 64
C = 4
M = 8
L = 256 * 64

BLK = 512
NBLK = B // BLK
LOG2PI = math.log(2.0 * math.pi)


def _te_block(inv_ref, oh_ref, xo_ref, xm_ref, nu_ref, tnu_ref, cinu_ref,
              ciwt_ref, ww_ref, tf_ref, aux_ref, ldets_ref, lp_ref,
              ci_ref, cicm_ref, out_ref, y1_ref, y2_ref, t_ref):
    inv_cap = inv_ref[0]
    xo = xo_ref[:]
    xm = xm_ref[:]
    nu_flat = nu_ref[:].reshape(BLK * C, DO)

    y1_ref[:] = jnp.zeros((BLK, DO), jnp.float32)
    y2_ref[:] = jnp.zeros((BLK, DM), jnp.float32)
    t_ref[:] = jnp.zeros((BLK * C, DM), jnp.float32)

    nbf = oh_ref[:]

    def h_body(h, carry):
        colmask = (nbf == h.astype(jnp.float32)).astype(jnp.float32)
        present = jnp.sum(colmask) > 0.5

        @pl.when(present)
        def _():
            ci_h = ci_ref[h]
            cicm_h = cicm_ref[h]
            y1_ref[:] += colmask * jnp.dot(
                xo, ci_h, preferred_element_type=jnp.float32)
            y2_ref[:] += colmask * jnp.dot(
                xo, cicm_h, preferred_element_type=jnp.float32)
            mask4 = jnp.broadcast_to(
                colmask[:, None, :], (BLK, C, 1)).reshape(BLK * C, 1)
            t_ref[:] += mask4 * jnp.dot(
                nu_flat, cicm_h, preferred_element_type=jnp.float32)

        return carry

    jax.lax.fori_loop(0, H, h_body, 0)

    y1 = y1_ref[:]
    y2 = y2_ref[:]
    t4 = t_ref[:].reshape(BLK, C, DM)

    ld = aux_ref[:, 0]
    nob = aux_ref[:, 1]
    nll = aux_ref[:, 2]
    xCx = jnp.sum(y1 * xo, axis=1)
    base = -0.5 * (ld + nob * LOG2PI) - nll

    nu = nu_ref[:]
    dx = xo[:, None, :] - nu
    mahal = xCx[:, None] - 2.0 * jnp.sum(xo[:, None, :] * cinu_ref[:], axis=2) \
        + jnp.sum(nu * cinu_ref[:], axis=2)

    p = jnp.sum(ciwt_ref[:] * dx[:, :, None, :], axis=3)

    jf = jax.lax.broadcasted_iota(jnp.int32, (M, M * M), 1)
    mrow = jax.lax.broadcasted_iota(jnp.int32, (M, M * M), 0)
    ra = (jf // M == mrow).astype(jnp.float32)
    rb = (jf % M == mrow).astype(jnp.float32)
    p2 = p.reshape(BLK * C, M)
    pa = jnp.dot(p2, ra, preferred_element_type=jnp.float32)
    pb = jnp.dot(p2, rb, preferred_element_type=jnp.float32)
    corr = jnp.sum(tf_ref[:].reshape(BLK * C, M * M) * pa * pb,
                   axis=1).reshape(BLK, C)

    r = xm[:, None, :] - tnu_ref[:] - y2[:, None, :] + t4
    rsq = jnp.sum(r * r, axis=2)

    jg = jax.lax.broadcasted_iota(jnp.int32, (DO, DO * M), 1)
    drow = jax.lax.broadcasted_iota(jnp.int32, (DO, DO * M), 0)
    rep_d = (jg // M == drow).astype(jnp.float32)
    jm = jax.lax.broadcasted_iota(jnp.int32, (M, DO * M), 1)
    mrow2 = jax.lax.broadcasted_iota(jnp.int32, (M, DO * M), 0)
    rep_m = (jm % M == mrow2).astype(jnp.float32)
    dxrep = jnp.dot(dx.reshape(BLK * C, DO), rep_d,
                    preferred_element_type=jnp.float32)
    prep = jnp.dot(p2, rep_m, preferred_element_type=jnp.float32)
    dp = dxrep * prep
    wq_p = jnp.sum(ww_ref[:].reshape(BLK * C, DO * M) * dp,
                   axis=1).reshape(BLK, C)

    lls = base[:, None] - 0.5 * (ldets_ref[:] + mahal - corr) + lp_ref[:]
    lls = lls - 0.5 * inv_cap * rsq
    lls = lls + wq_p
    out_ref[:] = lls


def kernel(batch_indices, features, neighborhood_ids, candidates,
           unit_neighb_lut, Coo_logdet, Coo_inv, Coinv_Com, obs_ix, miss_ix,
           nobs, log_proportions, nu, tnu, Wobs, Cooinv_nu, obs_logdets,
           Cobsinv_WobsT, T, W_WCC, inv_cap, noise_logliks):
    nb0 = neighborhood_ids[batch_indices]
    oh0 = jax.nn.one_hot(nb0, H, dtype=jnp.float32)

    iota_b = jnp.arange(B, dtype=jnp.int32)
    tril = (iota_b[:, None] >= iota_b[None, :]).astype(jnp.float32)
    cum = jnp.dot(tril, oh0, preferred_element_type=jnp.float32)
    rank = jnp.sum(cum * oh0, axis=1).astype(jnp.int32) - 1
    cnt = cum[-1].astype(jnp.int32)
    off = jnp.concatenate([jnp.zeros((1,), jnp.int32),
                           jnp.cumsum(cnt)[:-1]])
    pos = off[nb0] + rank
    inv = jnp.zeros((B,), jnp.int32).at[pos].set(iota_b)

    bi = batch_indices[inv]
    nb = nb0[inv]
    nbf_col = nb.astype(jnp.float32)[:, None]
    cand = candidates[bi]
    lut = unit_neighb_lut[cand, nb[:, None]]

    ft = jnp.swapaxes(features, 0, 1)
    xo = ft[obs_ix[nb], bi[:, None]]
    xm = ft[miss_ix[nb], bi[:, None]]

    nu_b = nu[lut]
    tnu_b = tnu[lut]
    cinu_b = Cooinv_nu[lut]
    ciwt_b = Cobsinv_WobsT[lut]
    comb = (W_WCC + 0.01 * Wobs).reshape(L, DO * M)
    ww_b = comb[lut]
    tf_b = T.reshape(L, M * M)[lut]

    ld = Coo_logdet[nb]
    ldets = obs_logdets[lut]
    lp = log_proportions[cand]
    nob = nobs[nb].astype(jnp.float32)
    nll = noise_logliks[bi]
    aux = jnp.stack([ld, nob, nll, jnp.zeros_like(ld)], axis=1)
    inv_arr = jnp.reshape(inv_cap, (1,)).astype(jnp.float32)

    spec = lambda bs, im: pl.BlockSpec(bs, im)
    in_specs = [
            pl.BlockSpec(memory_space=pltpu.SMEM),
            spec((BLK, 1), lambda i: (i, 0)),
            spec((BLK, DO), lambda i: (i, 0)),
            spec((BLK, DM), lambda i: (i, 0)),
            spec((BLK, C, DO), lambda i: (i, 0, 0)),
            spec((BLK, C, DM), lambda i: (i, 0, 0)),
            spec((BLK, C, DO), lambda i: (i, 0, 0)),
            spec((BLK, C, M, DO), lambda i: (i, 0, 0, 0)),
            spec((BLK, C, DO * M), lambda i: (i, 0, 0)),
            spec((BLK, C, M * M), lambda i: (i, 0, 0)),
            spec((BLK, 4), lambda i: (i, 0)),
            spec((BLK, C), lambda i: (i, 0)),
            spec((BLK, C), lambda i: (i, 0)),
            spec((H, DO, DO), lambda i: (0, 0, 0)),
            spec((H, DO, DM), lambda i: (0, 0, 0)),
    ]
    lls_sorted = pl.pallas_call(
        _te_block,
        grid=(NBLK,),
        in_specs=in_specs,
        out_specs=spec((BLK, C), lambda i: (i, 0)),
        out_shape=jax.ShapeDtypeStruct((B, C), jnp.float32),
        scratch_shapes=[
            pltpu.VMEM((BLK, DO), jnp.float32),
            pltpu.VMEM((BLK, DM), jnp.float32),
            pltpu.VMEM((BLK * C, DM), jnp.float32),
        ],
    )(inv_arr, nbf_col, xo, xm, nu_b, tnu_b, cinu_b, ciwt_b, ww_b, tf_b,
      aux, ldets, lp, Coo_inv, Coinv_Com)
    return lls_sorted[pos]
